# baseline (speedup 1.0000x reference)
import functools

import jax
import jax.numpy as jnp
from jax import lax
from jax.experimental import pallas as pl
from jax.experimental.pallas import tpu as pltpu
from jax.experimental.pallas import tpu_sc as plsc

N = 10000
D = 128
E = 320000
EL = 100000

NC = 2
NS = 16
NW = NC * NS

NP = 10240
RNG = NP // NS
DUMMYDL = RNG
ACC2R = RNG + 8

EH = E // NC
SCCH = 2000
NSC = EH // SCCH
ECAP = EH
LBUF = 4096
B2 = 128

SEC = 128
SCH = 25
SPT = SEC * SCH
ELP = NW * SPT

_MESH = plsc.VectorSubcoreMesh(core_axis_name="c", subcore_axis_name="s")
_SC_PARAMS = pltpu.CompilerParams(needs_layout_passes=False)


def _wid():
    return lax.axis_index("s") * NC + lax.axis_index("c")


@functools.partial(
    pl.kernel,
    out_type=[
        jax.ShapeDtypeStruct((NC, NP, 16), jnp.float32),
        jax.ShapeDtypeStruct((NW * ECAP,), jnp.int32),
        jax.ShapeDtypeStruct((NW * 16,), jnp.int32),
    ],
    mesh=_MESH,
    compiler_params=_SC_PARAMS,
    scratch_types=[
        pltpu.VMEM((SCCH,), jnp.int32),
        pltpu.VMEM((SCCH,), jnp.int32),
        pltpu.VMEM((LBUF,), jnp.int32),
        pltpu.VMEM((RNG, 16), jnp.float32),
        pltpu.VMEM((16,), jnp.int32),
        pltpu.VMEM((16,), jnp.int32),
        pltpu.SemaphoreType.DMA,
    ],
)
def _sc_route(dscan_hbm, sscan_hbm, degp_hbm, dlist_hbm, dcnt_hbm,
              dbuf, sbuf, lbuf, acc1, cbuf, stg, sem):
    c = lax.axis_index("c")
    s = lax.axis_index("s")
    wid = _wid()
    lane = lax.iota(jnp.int32, 16)
    ones = jnp.full((16,), 1.0, jnp.float32)

    def z1(i, carry):
        acc1[i, :] = jnp.zeros((16,), jnp.float32)
        return carry

    lax.fori_loop(0, RNG, z1, 0)

    base = c * EH
    lbase = wid * ECAP

    def chunk(k, carry):
        ptr, fo, tc, tail = carry
        pltpu.sync_copy(dscan_hbm.at[pl.ds(base + k * SCCH, SCCH)], dbuf)
        pltpu.sync_copy(sscan_hbm.at[pl.ds(base + k * SCCH, SCCH)], sbuf)

        def vec(i, c2):
            p, tc, tail = c2
            vd = dbuf[pl.ds(i * 16, 16)]
            vs = sbuf[pl.ds(i * 16, 16)]
            m2 = (vs & 15) == s
            plsc.addupdate_scatter(acc1, [vs >> 4, lane], ones, mask=m2)
            m1 = (vd & 15) == s
            cnt = plsc.all_reduce_population_count(m1)[0]
            plsc.store_compressed(stg.at[pl.ds(0, 16)], vd >> 4, mask=m1)
            cv = stg[pl.ds(0, 16)]
            rot = cv.at[(lane - tc) & 15].get(mode="promise_in_bounds")
            merged = jnp.where(lane >= tc, rot, tail)
            newfill = tc + cnt

            @pl.when(newfill >= 16)
            def _():
                lbuf[pl.ds(pl.multiple_of(p, 16), 16)] = merged

            p = p + jnp.where(newfill >= 16, 16, 0)
            tail = jnp.where(newfill >= 16, rot, merged)
            return p, newfill & 15, tail

        ptr, tc, tail = lax.fori_loop(0, SCCH // 16, vec, (ptr, tc, tail))

        flushed = jnp.where(ptr >= 2048, 1, 0)

        @pl.when(flushed == 1)
        def _():
            pltpu.sync_copy(lbuf.at[pl.ds(0, 2048)],
                            dlist_hbm.at[pl.ds(pl.multiple_of(lbase + fo, 128),
                                               2048)])
            for i in range(125):
                lbuf[pl.ds(i * 16, 16)] = lbuf[pl.ds(2048 + i * 16, 16)]

        return ptr - flushed * 2048, fo + flushed * 2048, tc, tail

    zero16 = jnp.zeros((16,), jnp.int32)
    ptr, fo, tc, tail = lax.fori_loop(0, NSC, chunk, (0, 0, 0, zero16))

    dummy = jnp.full((16,), DUMMYDL << 14, jnp.int32)

    @pl.when(tc > 0)
    def _():
        lbuf[pl.ds(pl.multiple_of(ptr, 16), 16)] =             jnp.where(lane < tc, tail, dummy)

    ptr = ptr + jnp.where(tc > 0, 16, 0)

    def pad(i, p):
        rem = p & 127

        @pl.when(rem != 0)
        def _():
            lbuf[pl.ds(pl.multiple_of(p, 16), 16)] = dummy

        return p + jnp.where(rem != 0, 16, 0)

    ptr = lax.fori_loop(0, 7, pad, ptr)

    nb = ptr >> 7

    def fl(i, carry):
        @pl.when(i < nb)
        def _():
            pltpu.sync_copy(
                lbuf.at[pl.ds(i * 128, 128)],
                dlist_hbm.at[pl.ds(pl.multiple_of(lbase + fo + i * 128, 128),
                                   128)])
        return carry

    lax.fori_loop(0, 32, fl, 0)

    total = fo + ptr
    cbuf[pl.ds(0, 16)] = jnp.zeros((16,), jnp.int32) + total
    pltpu.sync_copy(cbuf, dcnt_hbm.at[pl.ds(wid * 16, 16)])
    pltpu.sync_copy(acc1, degp_hbm.at[c].at[pl.ds(s * RNG, RNG)])


@functools.partial(
    pl.kernel,
    out_type=jax.ShapeDtypeStruct((NC * NP * D,), jnp.float32),
    mesh=_MESH,
    compiler_params=_SC_PARAMS,
    scratch_types=[
        pltpu.VMEM((B2, D), jnp.float32),
        pltpu.VMEM((B2,), jnp.int32),
        pltpu.VMEM((B2,), jnp.int32),
        pltpu.VMEM((ACC2R * D,), jnp.float32),
        pltpu.VMEM((16,), jnp.int32),
        pltpu.SemaphoreType.DMA,
    ],
)
def _sc_accum(hs_hbm, dlist_hbm, dcnt_hbm, out_hbm,
              rows_v, sidx, pbuf, acc2, cbuf, sem):
    c = lax.axis_index("c")
    s = lax.axis_index("s")
    wid = _wid()

    def z2(i, carry):
        for k in range(D // 16):
            acc2[pl.ds(i * D + k * 16, 16)] = jnp.zeros((16,), jnp.float32)
        return carry

    lax.fori_loop(0, ACC2R, z2, 0)

    pltpu.sync_copy(dcnt_hbm.at[pl.ds(wid * 16, 16)], cbuf)
    total = cbuf[pl.ds(0, 16)][0]
    nb = total >> 7
    lbase = wid * ECAP

    lane = lax.iota(jnp.int32, 16)
    cols = [k * 16 + lane for k in range(D // 16)]

    def batch(b, carry):
        off = pl.multiple_of(lbase + b * B2, 128)
        pltpu.sync_copy(dlist_hbm.at[pl.ds(off, B2)], pbuf)
        for i in range(B2 // 16):
            sidx[pl.ds(i * 16, 16)] = pbuf[pl.ds(i * 16, 16)] & 0x3FFF
        pltpu.async_copy(hs_hbm.at[sidx], rows_v, sem).wait()

        def group(g, c2):
            wbv = (pbuf[pl.ds(g * 16, 16)] >> 14) * D

            def edge(ee, c3):
                wb = wbv.at[jnp.zeros((16,), jnp.int32) + ee].get(
                    mode="promise_in_bounds")
                e = g * 16 + ee
                for k in range(D // 16):
                    plsc.addupdate_scatter(
                        acc2, [wb + cols[k]], rows_v[e, pl.ds(k * 16, 16)])
                return c3

            lax.fori_loop(0, 16, edge, 0)
            return c2

        lax.fori_loop(0, B2 // 16, group, 0)
        return carry

    lax.fori_loop(0, nb, batch, 0)
    obase = (c * NP + s * RNG) * D
    pltpu.sync_copy(acc2.at[pl.ds(0, RNG * D)],
                    out_hbm.at[pl.ds(pl.multiple_of(obase, 128), RNG * D)])


@functools.partial(
    pl.kernel,
    out_type=jax.ShapeDtypeStruct((ELP,), jnp.float32),
    mesh=_MESH,
    compiler_params=_SC_PARAMS,
    scratch_types=[
        pltpu.VMEM((SCH, SEC), jnp.int32),
        pltpu.VMEM((SCH, SEC), jnp.int32),
        pltpu.VMEM((SEC, D), jnp.float32),
        pltpu.VMEM((SEC, D), jnp.float32),
        pltpu.VMEM((SPT,), jnp.float32),
        pltpu.VMEM((16,), jnp.float32),
        pltpu.SemaphoreType.DMA,
    ],
)
def _sc_score(a_hbm, h_hbm, s_hbm, d_hbm, bsum_hbm, out_hbm,
              idx_s, idx_d, ra, rb, out_v, bsum_v, sem):
    wid = _wid()
    pltpu.sync_copy(s_hbm.at[wid], idx_s)
    pltpu.sync_copy(d_hbm.at[wid], idx_d)
    pltpu.sync_copy(bsum_hbm, bsum_v)
    bsum = bsum_v[pl.ds(0, 16)]
    lane = lax.iota(jnp.int32, 16)

    def chunk(j, carry):
        cp1 = pltpu.async_copy(a_hbm.at[idx_s.at[j]], ra, sem)
        cp2 = pltpu.async_copy(h_hbm.at[idx_d.at[j]], rb, sem)
        cp1.wait()
        cp2.wait()

        def group(g, carry2):
            erow = g * 16 + lane
            accs = [bsum, 0.0, 0.0, 0.0]
            for k in range(D):
                col = jnp.full((16,), k, jnp.int32)
                accs[k % 4] = accs[k % 4] + \
                    plsc.load_gather(ra, [erow, col]) * \
                    plsc.load_gather(rb, [erow, col])
            out_v[pl.ds(j * SEC + g * 16, 16)] = \
                (accs[0] + accs[1]) + (accs[2] + accs[3])
            return carry2

        lax.fori_loop(0, SEC // 16, group, 0)
        return carry

    lax.fori_loop(0, SCH, chunk, 0)
    pltpu.sync_copy(out_v, out_hbm.at[pl.ds(wid * SPT, SPT)])


def _tca_body(x_ref, wpret_ref, bpre_ref, degp_ref, h_ref, hs_ref, dinv_ref):
    h = jnp.dot(x_ref[...], wpret_ref[...],
                preferred_element_type=jnp.float32) + bpre_ref[...]
    deg = jnp.sum(degp_ref[0] + degp_ref[1], axis=-1, keepdims=True)
    dinv = jnp.where(deg > 0, lax.rsqrt(deg), 0.0)
    h_ref[...] = h
    hs_ref[...] = h * dinv
    dinv_ref[...] = dinv


def _tc_pre(x, wpret, bpre_r, degp):
    bn = 1000
    grid = N // bn
    return pl.pallas_call(
        _tca_body,
        grid=(grid,),
        in_specs=[
            pl.BlockSpec((bn, D), lambda i: (i, 0)),
            pl.BlockSpec((D, D), lambda i: (0, 0)),
            pl.BlockSpec((1, D), lambda i: (0, 0)),
            pl.BlockSpec((NC, bn, 16), lambda i: (0, i, 0)),
        ],
        out_specs=[
            pl.BlockSpec((bn, D), lambda i: (i, 0)),
            pl.BlockSpec((bn, D), lambda i: (i, 0)),
            pl.BlockSpec((bn, 1), lambda i: (i, 0)),
        ],
        out_shape=[
            jax.ShapeDtypeStruct((N, D), jnp.float32),
            jax.ShapeDtypeStruct((N, D), jnp.float32),
            jax.ShapeDtypeStruct((N, 1), jnp.float32),
        ],
    )(x, wpret, bpre_r, degp)


def _tcb_body(h_ref, tp_ref, dinv_ref, wz0_ref, wz1_ref, wh0_ref, wh1_ref,
              bz_ref, bh_ref, wsum_ref, hr_ref, a_ref):
    h = h_ref[...]
    u = dinv_ref[...] * (tp_ref[0] + tp_ref[1])
    z = jax.nn.sigmoid(
        jnp.dot(h, wz0_ref[...], preferred_element_type=jnp.float32)
        - jnp.dot(u, wz1_ref[...], preferred_element_type=jnp.float32)
        + bz_ref[...])
    ht = jnp.tanh(
        jnp.dot(h, wh0_ref[...], preferred_element_type=jnp.float32)
        - jnp.dot(u, wh1_ref[...], preferred_element_type=jnp.float32)
        + bh_ref[...])
    hr = jnp.maximum((1.0 - z) * ht, 0.0)
    hr_ref[...] = hr
    a_ref[...] = hr * wsum_ref[...]


def _tc_gates(h, tp, dinv, wz0, wz1, wh0, wh1, bz_r, bh_r, wsum_r):
    bn = 1000
    grid = N // bn
    wspec = pl.BlockSpec((D, D), lambda i: (0, 0))
    bspec = pl.BlockSpec((1, D), lambda i: (0, 0))
    return pl.pallas_call(
        _tcb_body,
        grid=(grid,),
        in_specs=[
            pl.BlockSpec((bn, D), lambda i: (i, 0)),
            pl.BlockSpec((NC, bn, D), lambda i: (0, i, 0)),
            pl.BlockSpec((bn, 1), lambda i: (i, 0)),
            wspec, wspec, wspec, wspec, bspec, bspec, bspec,
        ],
        out_specs=[
            pl.BlockSpec((bn, D), lambda i: (i, 0)),
            pl.BlockSpec((bn, D), lambda i: (i, 0)),
        ],
        out_shape=[
            jax.ShapeDtypeStruct((N, D), jnp.float32),
            jax.ShapeDtypeStruct((N, D), jnp.float32),
        ],
    )(h, tp, dinv, wz0, wz1, wh0, wh1, bz_r, bh_r, wsum_r)


def kernel(x, edge_index, edge_label_index, Wpre, bpre,
           xz_W0, xz_W1, xz_b, hz_W0, hz_W1, hz_b,
           xr_W0, xr_W1, xr_b, hr_W0, hr_W1, hr_b,
           xh_W0, xh_W1, xh_b, hh_W0, hh_W1, hh_b,
           Wpost, bpost):
    src = edge_index[0]
    dst = edge_index[1]
    dscan = (dst // RNG) | (src << 4) | ((dst % RNG) << 18)
    sscan = (src // RNG) | ((src % RNG) << 4)

    degp, dlist, dcnt = _sc_route(dscan, sscan)
    h, hs, dinv = _tc_pre(x, Wpre.T, bpre[None, :], degp[:, :N, :])
    tp = _sc_accum(hs, dlist, dcnt).reshape(NC, NP, D)[:, :N, :]
    hrelu, a = _tc_gates(
        h, tp, dinv, xz_W0, xz_W1, xh_W0, xh_W1,
        (xz_b + hz_b)[None, :], (xh_b + hh_b)[None, :],
        (Wpost[0] + Wpost[1])[None, :])

    eli = jnp.concatenate(
        [edge_label_index,
         jnp.zeros((2, ELP - EL), dtype=edge_label_index.dtype)], axis=1)
    s_r = eli[0].reshape(NW, SCH, SEC)
    d_r = eli[1].reshape(NW, SCH, SEC)
    bsum_arr = jnp.full((16,), bpost[0] + bpost[1], dtype=jnp.float32)

    scores = _sc_score(a, hrelu, s_r, d_r, bsum_arr)
    return scores[:EL]


# speedup vs baseline: 1.1678x; 1.1678x over previous
import functools

import jax
import jax.numpy as jnp
from jax import lax
from jax.experimental import pallas as pl
from jax.experimental.pallas import tpu as pltpu
from jax.experimental.pallas import tpu_sc as plsc

N = 10000
D = 128
E = 320000
EL = 100000

NC = 2
NS = 16
NW = NC * NS

NP = 10240
RNG = NP // NS
DUMMYDL = RNG
ACC2R = RNG + 8

EH = E // NC
SCCH = 2000
NSC = EH // SCCH
ECAP = EH
LBUF = 4096
B2 = 128

SEC = 128
SCH = 26
SPT = SEC * SCH
ELP = NW * SPT

_MESH = plsc.VectorSubcoreMesh(core_axis_name="c", subcore_axis_name="s")
_SC_PARAMS = pltpu.CompilerParams(needs_layout_passes=False)


def _wid():
    return lax.axis_index("s") * NC + lax.axis_index("c")


@functools.partial(
    pl.kernel,
    out_type=[
        jax.ShapeDtypeStruct((NC, NP, 16), jnp.float32),
        jax.ShapeDtypeStruct((NW * ECAP,), jnp.int32),
        jax.ShapeDtypeStruct((NW * 16,), jnp.int32),
    ],
    mesh=_MESH,
    compiler_params=_SC_PARAMS,
    scratch_types=[
        pltpu.VMEM((SCCH,), jnp.int32),
        pltpu.VMEM((SCCH,), jnp.int32),
        pltpu.VMEM((LBUF,), jnp.int32),
        pltpu.VMEM((RNG, 16), jnp.float32),
        pltpu.VMEM((16,), jnp.int32),
        pltpu.VMEM((16,), jnp.int32),
        pltpu.SemaphoreType.DMA,
    ],
)
def _sc_route(dscan_hbm, sscan_hbm, degp_hbm, dlist_hbm, dcnt_hbm,
              dbuf, sbuf, lbuf, acc1, cbuf, stg, sem):
    c = lax.axis_index("c")
    s = lax.axis_index("s")
    wid = _wid()
    lane = lax.iota(jnp.int32, 16)
    ones = jnp.full((16,), 1.0, jnp.float32)

    def z1(i, carry):
        acc1[i, :] = jnp.zeros((16,), jnp.float32)
        return carry

    lax.fori_loop(0, RNG, z1, 0)

    base = c * EH
    lbase = wid * ECAP

    def chunk(k, carry):
        ptr, fo, tc, tail = carry
        pltpu.sync_copy(dscan_hbm.at[pl.ds(base + k * SCCH, SCCH)], dbuf)
        pltpu.sync_copy(sscan_hbm.at[pl.ds(base + k * SCCH, SCCH)], sbuf)

        def vec(i, c2):
            p, tc, tail = c2
            vd = dbuf[pl.ds(i * 16, 16)]
            vs = sbuf[pl.ds(i * 16, 16)]
            m2 = (vs & 15) == s
            plsc.addupdate_scatter(acc1, [vs >> 4, lane], ones, mask=m2)
            m1 = (vd & 15) == s
            cnt = plsc.all_reduce_population_count(m1)[0]
            plsc.store_compressed(stg.at[pl.ds(0, 16)], vd >> 4, mask=m1)
            cv = stg[pl.ds(0, 16)]
            rot = cv.at[(lane - tc) & 15].get(mode="promise_in_bounds")
            merged = jnp.where(lane >= tc, rot, tail)
            newfill = tc + cnt

            @pl.when(newfill >= 16)
            def _():
                lbuf[pl.ds(pl.multiple_of(p, 16), 16)] = merged

            p = p + jnp.where(newfill >= 16, 16, 0)
            tail = jnp.where(newfill >= 16, rot, merged)
            return p, newfill & 15, tail

        ptr, tc, tail = lax.fori_loop(0, SCCH // 16, vec, (ptr, tc, tail))

        flushed = jnp.where(ptr >= 2048, 1, 0)

        @pl.when(flushed == 1)
        def _():
            pltpu.sync_copy(lbuf.at[pl.ds(0, 2048)],
                            dlist_hbm.at[pl.ds(pl.multiple_of(lbase + fo, 128),
                                               2048)])
            for i in range(125):
                lbuf[pl.ds(i * 16, 16)] = lbuf[pl.ds(2048 + i * 16, 16)]

        return ptr - flushed * 2048, fo + flushed * 2048, tc, tail

    zero16 = jnp.zeros((16,), jnp.int32)
    ptr, fo, tc, tail = lax.fori_loop(0, NSC, chunk, (0, 0, 0, zero16))

    dummy = jnp.full((16,), DUMMYDL << 14, jnp.int32)

    @pl.when(tc > 0)
    def _():
        lbuf[pl.ds(pl.multiple_of(ptr, 16), 16)] =             jnp.where(lane < tc, tail, dummy)

    ptr = ptr + jnp.where(tc > 0, 16, 0)

    def pad(i, p):
        rem = p & 127

        @pl.when(rem != 0)
        def _():
            lbuf[pl.ds(pl.multiple_of(p, 16), 16)] = dummy

        return p + jnp.where(rem != 0, 16, 0)

    ptr = lax.fori_loop(0, 7, pad, ptr)

    nb = ptr >> 7

    def fl(i, carry):
        @pl.when(i < nb)
        def _():
            pltpu.sync_copy(
                lbuf.at[pl.ds(i * 128, 128)],
                dlist_hbm.at[pl.ds(pl.multiple_of(lbase + fo + i * 128, 128),
                                   128)])
        return carry

    lax.fori_loop(0, 32, fl, 0)

    total = fo + ptr
    cbuf[pl.ds(0, 16)] = jnp.zeros((16,), jnp.int32) + total
    pltpu.sync_copy(cbuf, dcnt_hbm.at[pl.ds(wid * 16, 16)])
    pltpu.sync_copy(acc1, degp_hbm.at[c].at[pl.ds(s * RNG, RNG)])


@functools.partial(
    pl.kernel,
    out_type=jax.ShapeDtypeStruct((NC * NP * D,), jnp.float32),
    mesh=_MESH,
    compiler_params=_SC_PARAMS,
    scratch_types=[
        pltpu.VMEM((B2, D), jnp.float32),
        pltpu.VMEM((B2,), jnp.int32),
        pltpu.VMEM((B2,), jnp.int32),
        pltpu.VMEM((ACC2R * D,), jnp.float32),
        pltpu.VMEM((16,), jnp.int32),
        pltpu.SemaphoreType.DMA,
    ],
)
def _sc_accum(hs_hbm, dlist_hbm, dcnt_hbm, out_hbm,
              rows_v, sidx, pbuf, acc2, cbuf, sem):
    c = lax.axis_index("c")
    s = lax.axis_index("s")
    wid = _wid()

    def z2(i, carry):
        for k in range(D // 16):
            acc2[pl.ds(i * D + k * 16, 16)] = jnp.zeros((16,), jnp.float32)
        return carry

    lax.fori_loop(0, ACC2R, z2, 0)

    pltpu.sync_copy(dcnt_hbm.at[pl.ds(wid * 16, 16)], cbuf)
    total = cbuf[pl.ds(0, 16)][0]
    nb = total >> 7
    lbase = wid * ECAP

    lane = lax.iota(jnp.int32, 16)
    cols = [k * 16 + lane for k in range(D // 16)]

    def batch(b, carry):
        off = pl.multiple_of(lbase + b * B2, 128)
        pltpu.sync_copy(dlist_hbm.at[pl.ds(off, B2)], pbuf)
        for i in range(B2 // 16):
            sidx[pl.ds(i * 16, 16)] = pbuf[pl.ds(i * 16, 16)] & 0x3FFF
        pltpu.async_copy(hs_hbm.at[sidx], rows_v, sem).wait()

        def group(g, c2):
            wbv = (pbuf[pl.ds(g * 16, 16)] >> 14) * D

            def edge(ee, c3):
                wb = wbv.at[jnp.zeros((16,), jnp.int32) + ee].get(
                    mode="promise_in_bounds")
                e = g * 16 + ee
                for k in range(D // 16):
                    plsc.addupdate_scatter(
                        acc2, [wb + cols[k]], rows_v[e, pl.ds(k * 16, 16)])
                return c3

            lax.fori_loop(0, 16, edge, 0)
            return c2

        lax.fori_loop(0, B2 // 16, group, 0)
        return carry

    lax.fori_loop(0, nb, batch, 0)
    obase = (c * NP + s * RNG) * D
    pltpu.sync_copy(acc2.at[pl.ds(0, RNG * D)],
                    out_hbm.at[pl.ds(pl.multiple_of(obase, 128), RNG * D)])


@functools.partial(
    pl.kernel,
    out_type=jax.ShapeDtypeStruct((ELP,), jnp.float32),
    mesh=_MESH,
    compiler_params=_SC_PARAMS,
    scratch_types=[
        pltpu.VMEM((SCH, SEC), jnp.int32),
        pltpu.VMEM((SCH, SEC), jnp.int32),
        pltpu.VMEM((SEC, D), jnp.float32),
        pltpu.VMEM((SEC, D), jnp.float32),
        pltpu.VMEM((SEC, D), jnp.float32),
        pltpu.VMEM((SEC, D), jnp.float32),
        pltpu.VMEM((SPT,), jnp.float32),
        pltpu.VMEM((16,), jnp.float32),
        pltpu.SemaphoreType.DMA,
        pltpu.SemaphoreType.DMA,
    ],
)
def _sc_score(a_hbm, h_hbm, s_hbm, d_hbm, bsum_hbm, out_hbm,
              idx_s, idx_d, ra0, rb0, ra1, rb1, out_v, bsum_v, sem0, sem1):
    wid = _wid()
    pltpu.sync_copy(s_hbm.at[wid], idx_s)
    pltpu.sync_copy(d_hbm.at[wid], idx_d)
    pltpu.sync_copy(bsum_hbm, bsum_v)
    bsum = bsum_v[pl.ds(0, 16)]
    lane = lax.iota(jnp.int32, 16)

    def start(j, ra, rb, sem):
        pltpu.async_copy(a_hbm.at[idx_s.at[j]], ra, sem)
        pltpu.async_copy(h_hbm.at[idx_d.at[j]], rb, sem)

    def drain(ra, rb, sem):
        pltpu.make_async_copy(a_hbm.at[idx_s.at[0]], ra, sem).wait()
        pltpu.make_async_copy(h_hbm.at[idx_d.at[0]], rb, sem).wait()

    def process(j, ra, rb):
        def group(g, carry2):
            erow = g * 16 + lane
            acc = bsum
            for k in range(D):
                col = jnp.full((16,), k, jnp.int32)
                acc = acc + plsc.load_gather(ra, [erow, col]) * \
                    plsc.load_gather(rb, [erow, col])
            out_v[pl.ds(j * SEC + g * 16, 16)] = acc
            return carry2

        lax.fori_loop(0, SEC // 16, group, 0)

    start(0, ra0, rb0, sem0)

    def pair(i, carry):
        ja = 2 * i
        start(ja + 1, ra1, rb1, sem1)
        drain(ra0, rb0, sem0)
        process(ja, ra0, rb0)

        @pl.when(i < SCH // 2 - 1)
        def _():
            start(ja + 2, ra0, rb0, sem0)

        drain(ra1, rb1, sem1)
        process(ja + 1, ra1, rb1)
        return carry

    lax.fori_loop(0, SCH // 2, pair, 0)
    pltpu.sync_copy(out_v, out_hbm.at[pl.ds(wid * SPT, SPT)])


def _tca_body(x_ref, wpret_ref, bpre_ref, degp_ref, h_ref, hs_ref, dinv_ref):
    h = jnp.dot(x_ref[...], wpret_ref[...],
                preferred_element_type=jnp.float32) + bpre_ref[...]
    deg = jnp.sum(degp_ref[0] + degp_ref[1], axis=-1, keepdims=True)
    dinv = jnp.where(deg > 0, lax.rsqrt(deg), 0.0)
    h_ref[...] = h
    hs_ref[...] = h * dinv
    dinv_ref[...] = dinv


def _tc_pre(x, wpret, bpre_r, degp):
    bn = 1000
    grid = N // bn
    return pl.pallas_call(
        _tca_body,
        grid=(grid,),
        in_specs=[
            pl.BlockSpec((bn, D), lambda i: (i, 0)),
            pl.BlockSpec((D, D), lambda i: (0, 0)),
            pl.BlockSpec((1, D), lambda i: (0, 0)),
            pl.BlockSpec((NC, bn, 16), lambda i: (0, i, 0)),
        ],
        out_specs=[
            pl.BlockSpec((bn, D), lambda i: (i, 0)),
            pl.BlockSpec((bn, D), lambda i: (i, 0)),
            pl.BlockSpec((bn, 1), lambda i: (i, 0)),
        ],
        out_shape=[
            jax.ShapeDtypeStruct((N, D), jnp.float32),
            jax.ShapeDtypeStruct((N, D), jnp.float32),
            jax.ShapeDtypeStruct((N, 1), jnp.float32),
        ],
    )(x, wpret, bpre_r, degp)


def _tcb_body(h_ref, tp_ref, dinv_ref, wz0_ref, wz1_ref, wh0_ref, wh1_ref,
              bz_ref, bh_ref, wsum_ref, hr_ref, a_ref):
    h = h_ref[...]
    u = dinv_ref[...] * (tp_ref[0] + tp_ref[1])
    z = jax.nn.sigmoid(
        jnp.dot(h, wz0_ref[...], preferred_element_type=jnp.float32)
        - jnp.dot(u, wz1_ref[...], preferred_element_type=jnp.float32)
        + bz_ref[...])
    ht = jnp.tanh(
        jnp.dot(h, wh0_ref[...], preferred_element_type=jnp.float32)
        - jnp.dot(u, wh1_ref[...], preferred_element_type=jnp.float32)
        + bh_ref[...])
    hr = jnp.maximum((1.0 - z) * ht, 0.0)
    hr_ref[...] = hr
    a_ref[...] = hr * wsum_ref[...]


def _tc_gates(h, tp, dinv, wz0, wz1, wh0, wh1, bz_r, bh_r, wsum_r):
    bn = 1000
    grid = N // bn
    wspec = pl.BlockSpec((D, D), lambda i: (0, 0))
    bspec = pl.BlockSpec((1, D), lambda i: (0, 0))
    return pl.pallas_call(
        _tcb_body,
        grid=(grid,),
        in_specs=[
            pl.BlockSpec((bn, D), lambda i: (i, 0)),
            pl.BlockSpec((NC, bn, D), lambda i: (0, i, 0)),
            pl.BlockSpec((bn, 1), lambda i: (i, 0)),
            wspec, wspec, wspec, wspec, bspec, bspec, bspec,
        ],
        out_specs=[
            pl.BlockSpec((bn, D), lambda i: (i, 0)),
            pl.BlockSpec((bn, D), lambda i: (i, 0)),
        ],
        out_shape=[
            jax.ShapeDtypeStruct((N, D), jnp.float32),
            jax.ShapeDtypeStruct((N, D), jnp.float32),
        ],
    )(h, tp, dinv, wz0, wz1, wh0, wh1, bz_r, bh_r, wsum_r)


def kernel(x, edge_index, edge_label_index, Wpre, bpre,
           xz_W0, xz_W1, xz_b, hz_W0, hz_W1, hz_b,
           xr_W0, xr_W1, xr_b, hr_W0, hr_W1, hr_b,
           xh_W0, xh_W1, xh_b, hh_W0, hh_W1, hh_b,
           Wpost, bpost):
    src = edge_index[0]
    dst = edge_index[1]
    dscan = (dst // RNG) | (src << 4) | ((dst % RNG) << 18)
    sscan = (src // RNG) | ((src % RNG) << 4)

    degp, dlist, dcnt = _sc_route(dscan, sscan)
    h, hs, dinv = _tc_pre(x, Wpre.T, bpre[None, :], degp[:, :N, :])
    tp = _sc_accum(hs, dlist, dcnt).reshape(NC, NP, D)[:, :N, :]
    hrelu, a = _tc_gates(
        h, tp, dinv, xz_W0, xz_W1, xh_W0, xh_W1,
        (xz_b + hz_b)[None, :], (xh_b + hh_b)[None, :],
        (Wpost[0] + Wpost[1])[None, :])

    eli = jnp.concatenate(
        [edge_label_index,
         jnp.zeros((2, ELP - EL), dtype=edge_label_index.dtype)], axis=1)
    s_r = eli[0].reshape(NW, SCH, SEC)
    d_r = eli[1].reshape(NW, SCH, SEC)
    bsum_arr = jnp.full((16,), bpost[0] + bpost[1], dtype=jnp.float32)

    scores = _sc_score(a, hrelu, s_r, d_r, bsum_arr)
    return scores[:EL]


# speedup vs baseline: 1.2623x; 1.0809x over previous
import functools

import jax
import jax.numpy as jnp
from jax import lax
from jax.experimental import pallas as pl
from jax.experimental.pallas import tpu as pltpu
from jax.experimental.pallas import tpu_sc as plsc

N = 10000
D = 128
E = 320000
EL = 100000

NC = 2
NS = 16
NW = NC * NS

NP = 10240
RNG = NP // NS
DUMMYDL = RNG
ACC2R = RNG + 8

EH = E // NC
SCCH = 2000
NSC = EH // SCCH
ECAP = EH
LBUF = 4096
B2 = 128

SEC = 128
SCH = 26
SPT = SEC * SCH
ELP = NW * SPT

_MESH = plsc.VectorSubcoreMesh(core_axis_name="c", subcore_axis_name="s")
_SC_PARAMS = pltpu.CompilerParams(needs_layout_passes=False)


def _wid():
    return lax.axis_index("s") * NC + lax.axis_index("c")


@functools.partial(
    pl.kernel,
    out_type=[
        jax.ShapeDtypeStruct((NC, NP, 16), jnp.float32),
        jax.ShapeDtypeStruct((NW * ECAP,), jnp.int32),
        jax.ShapeDtypeStruct((NW * 16,), jnp.int32),
    ],
    mesh=_MESH,
    compiler_params=_SC_PARAMS,
    scratch_types=[
        pltpu.VMEM((SCCH,), jnp.int32),
        pltpu.VMEM((SCCH,), jnp.int32),
        pltpu.VMEM((LBUF,), jnp.int32),
        pltpu.VMEM((RNG, 16), jnp.float32),
        pltpu.VMEM((16,), jnp.int32),
        pltpu.VMEM((16,), jnp.int32),
        pltpu.SemaphoreType.DMA,
    ],
)
def _sc_route(dscan_hbm, sscan_hbm, degp_hbm, dlist_hbm, dcnt_hbm,
              dbuf, sbuf, lbuf, acc1, cbuf, stg, sem):
    c = lax.axis_index("c")
    s = lax.axis_index("s")
    wid = _wid()
    lane = lax.iota(jnp.int32, 16)
    ones = jnp.full((16,), 1.0, jnp.float32)

    def z1(i, carry):
        acc1[i, :] = jnp.zeros((16,), jnp.float32)
        return carry

    lax.fori_loop(0, RNG, z1, 0)

    base = c * EH
    lbase = wid * ECAP

    def chunk(k, carry):
        ptr, fo, tc, tail = carry
        pltpu.sync_copy(dscan_hbm.at[pl.ds(base + k * SCCH, SCCH)], dbuf)
        pltpu.sync_copy(sscan_hbm.at[pl.ds(base + k * SCCH, SCCH)], sbuf)

        def vec(i, c2):
            p, tc, tail = c2
            vd = dbuf[pl.ds(i * 16, 16)]
            vs = sbuf[pl.ds(i * 16, 16)]
            m2 = (vs & 15) == s
            plsc.addupdate_scatter(acc1, [vs >> 4, lane], ones, mask=m2)
            m1 = (vd & 15) == s
            cnt = plsc.all_reduce_population_count(m1)[0]
            plsc.store_compressed(stg.at[pl.ds(0, 16)], vd >> 4, mask=m1)
            cv = stg[pl.ds(0, 16)]
            rot = cv.at[(lane - tc) & 15].get(mode="promise_in_bounds")
            merged = jnp.where(lane >= tc, rot, tail)
            newfill = tc + cnt

            @pl.when(newfill >= 16)
            def _():
                lbuf[pl.ds(pl.multiple_of(p, 16), 16)] = merged

            p = p + jnp.where(newfill >= 16, 16, 0)
            tail = jnp.where(newfill >= 16, rot, merged)
            return p, newfill & 15, tail

        ptr, tc, tail = lax.fori_loop(0, SCCH // 16, vec, (ptr, tc, tail))

        flushed = jnp.where(ptr >= 2048, 1, 0)

        @pl.when(flushed == 1)
        def _():
            pltpu.sync_copy(lbuf.at[pl.ds(0, 2048)],
                            dlist_hbm.at[pl.ds(pl.multiple_of(lbase + fo, 128),
                                               2048)])
            for i in range(125):
                lbuf[pl.ds(i * 16, 16)] = lbuf[pl.ds(2048 + i * 16, 16)]

        return ptr - flushed * 2048, fo + flushed * 2048, tc, tail

    zero16 = jnp.zeros((16,), jnp.int32)
    ptr, fo, tc, tail = lax.fori_loop(0, NSC, chunk, (0, 0, 0, zero16))

    dummy = jnp.full((16,), DUMMYDL << 14, jnp.int32)

    @pl.when(tc > 0)
    def _():
        lbuf[pl.ds(pl.multiple_of(ptr, 16), 16)] =             jnp.where(lane < tc, tail, dummy)

    ptr = ptr + jnp.where(tc > 0, 16, 0)

    def pad(i, p):
        rem = p & 127

        @pl.when(rem != 0)
        def _():
            lbuf[pl.ds(pl.multiple_of(p, 16), 16)] = dummy

        return p + jnp.where(rem != 0, 16, 0)

    ptr = lax.fori_loop(0, 7, pad, ptr)

    nb = ptr >> 7

    def fl(i, carry):
        @pl.when(i < nb)
        def _():
            pltpu.sync_copy(
                lbuf.at[pl.ds(i * 128, 128)],
                dlist_hbm.at[pl.ds(pl.multiple_of(lbase + fo + i * 128, 128),
                                   128)])
        return carry

    lax.fori_loop(0, 32, fl, 0)

    total = fo + ptr
    cbuf[pl.ds(0, 16)] = jnp.zeros((16,), jnp.int32) + total
    pltpu.sync_copy(cbuf, dcnt_hbm.at[pl.ds(wid * 16, 16)])
    pltpu.sync_copy(acc1, degp_hbm.at[c].at[pl.ds(s * RNG, RNG)])


@functools.partial(
    pl.kernel,
    out_type=jax.ShapeDtypeStruct((NC * NP * D,), jnp.float32),
    mesh=_MESH,
    compiler_params=_SC_PARAMS,
    scratch_types=[
        pltpu.VMEM((B2, D), jnp.float32),
        pltpu.VMEM((B2, D), jnp.float32),
        pltpu.VMEM((B2,), jnp.int32),
        pltpu.VMEM((B2,), jnp.int32),
        pltpu.VMEM((B2,), jnp.int32),
        pltpu.VMEM((B2,), jnp.int32),
        pltpu.VMEM((ACC2R * D,), jnp.float32),
        pltpu.VMEM((16,), jnp.int32),
        pltpu.SemaphoreType.DMA,
        pltpu.SemaphoreType.DMA,
    ],
)
def _sc_accum(hs_hbm, dlist_hbm, dcnt_hbm, out_hbm,
              rows0, rows1, sidx0, sidx1, pbuf0, pbuf1, acc2, cbuf,
              sem0, sem1):
    c = lax.axis_index("c")
    s = lax.axis_index("s")
    wid = _wid()

    def z2(i, carry):
        for k in range(D // 16):
            acc2[pl.ds(i * D + k * 16, 16)] = jnp.zeros((16,), jnp.float32)
        return carry

    lax.fori_loop(0, ACC2R, z2, 0)

    pltpu.sync_copy(dcnt_hbm.at[pl.ds(wid * 16, 16)], cbuf)
    total = cbuf[pl.ds(0, 16)][0]
    nb = total >> 7
    lbase = wid * ECAP

    lane = lax.iota(jnp.int32, 16)
    cols = [k * 16 + lane for k in range(D // 16)]

    def fetch(b, pbuf, sidx, rows, sem):
        off = pl.multiple_of(lbase + b * B2, 128)
        pltpu.sync_copy(dlist_hbm.at[pl.ds(off, B2)], pbuf)
        for i in range(B2 // 16):
            sidx[pl.ds(i * 16, 16)] = pbuf[pl.ds(i * 16, 16)] & 0x3FFF
        pltpu.async_copy(hs_hbm.at[sidx], rows, sem)

    def drain(sidx, rows, sem):
        pltpu.make_async_copy(hs_hbm.at[sidx], rows, sem).wait()

    def process(pbuf, rows):
        def group(g, c2):
            wbv = (pbuf[pl.ds(g * 16, 16)] >> 14) * D

            def edge(ee, c3):
                wb = wbv.at[jnp.zeros((16,), jnp.int32) + ee].get(
                    mode="promise_in_bounds")
                e = g * 16 + ee
                for k in range(D // 16):
                    plsc.addupdate_scatter(
                        acc2, [wb + cols[k]], rows[e, pl.ds(k * 16, 16)])
                return c3

            lax.fori_loop(0, 16, edge, 0)
            return c2

        lax.fori_loop(0, B2 // 16, group, 0)

    @pl.when(nb > 0)
    def _():
        fetch(0, pbuf0, sidx0, rows0, sem0)

    def pair(i, carry):
        ja = 2 * i

        @pl.when(ja + 1 < nb)
        def _():
            fetch(ja + 1, pbuf1, sidx1, rows1, sem1)

        drain(sidx0, rows0, sem0)
        process(pbuf0, rows0)

        @pl.when(ja + 2 < nb)
        def _():
            fetch(ja + 2, pbuf0, sidx0, rows0, sem0)

        @pl.when(ja + 1 < nb)
        def _():
            drain(sidx1, rows1, sem1)
            process(pbuf1, rows1)

        return carry

    lax.fori_loop(0, (nb + 1) >> 1, pair, 0)
    obase = (c * NP + s * RNG) * D
    pltpu.sync_copy(acc2.at[pl.ds(0, RNG * D)],
                    out_hbm.at[pl.ds(pl.multiple_of(obase, 128), RNG * D)])


@functools.partial(
    pl.kernel,
    out_type=jax.ShapeDtypeStruct((ELP,), jnp.float32),
    mesh=_MESH,
    compiler_params=_SC_PARAMS,
    scratch_types=[
        pltpu.VMEM((SCH, SEC), jnp.int32),
        pltpu.VMEM((SCH, SEC), jnp.int32),
        pltpu.VMEM((SEC, D), jnp.float32),
        pltpu.VMEM((SEC, D), jnp.float32),
        pltpu.VMEM((SEC, D), jnp.float32),
        pltpu.VMEM((SEC, D), jnp.float32),
        pltpu.VMEM((SPT,), jnp.float32),
        pltpu.VMEM((16,), jnp.float32),
        pltpu.SemaphoreType.DMA,
        pltpu.SemaphoreType.DMA,
    ],
)
def _sc_score(a_hbm, h_hbm, s_hbm, d_hbm, bsum_hbm, out_hbm,
              idx_s, idx_d, ra0, rb0, ra1, rb1, out_v, bsum_v, sem0, sem1):
    wid = _wid()
    pltpu.sync_copy(s_hbm.at[wid], idx_s)
    pltpu.sync_copy(d_hbm.at[wid], idx_d)
    pltpu.sync_copy(bsum_hbm, bsum_v)
    bsum = bsum_v[pl.ds(0, 16)]
    lane = lax.iota(jnp.int32, 16)

    def start(j, ra, rb, sem):
        pltpu.async_copy(a_hbm.at[idx_s.at[j]], ra, sem)
        pltpu.async_copy(h_hbm.at[idx_d.at[j]], rb, sem)

    def drain(ra, rb, sem):
        pltpu.make_async_copy(a_hbm.at[idx_s.at[0]], ra, sem).wait()
        pltpu.make_async_copy(h_hbm.at[idx_d.at[0]], rb, sem).wait()

    def process(j, ra, rb):
        def group(g, carry2):
            erow = g * 16 + lane
            acc = bsum
            for k in range(D):
                col = jnp.full((16,), k, jnp.int32)
                acc = acc + plsc.load_gather(ra, [erow, col]) * \
                    plsc.load_gather(rb, [erow, col])
            out_v[pl.ds(j * SEC + g * 16, 16)] = acc
            return carry2

        lax.fori_loop(0, SEC // 16, group, 0)

    start(0, ra0, rb0, sem0)

    def pair(i, carry):
        ja = 2 * i
        start(ja + 1, ra1, rb1, sem1)
        drain(ra0, rb0, sem0)
        process(ja, ra0, rb0)

        @pl.when(i < SCH // 2 - 1)
        def _():
            start(ja + 2, ra0, rb0, sem0)

        drain(ra1, rb1, sem1)
        process(ja + 1, ra1, rb1)
        return carry

    lax.fori_loop(0, SCH // 2, pair, 0)
    pltpu.sync_copy(out_v, out_hbm.at[pl.ds(wid * SPT, SPT)])


def _tca_body(x_ref, wpret_ref, bpre_ref, degp_ref, h_ref, hs_ref, dinv_ref):
    h = jnp.dot(x_ref[...], wpret_ref[...],
                preferred_element_type=jnp.float32) + bpre_ref[...]
    deg = jnp.sum(degp_ref[0] + degp_ref[1], axis=-1, keepdims=True)
    dinv = jnp.where(deg > 0, lax.rsqrt(deg), 0.0)
    h_ref[...] = h
    hs_ref[...] = h * dinv
    dinv_ref[...] = dinv


def _tc_pre(x, wpret, bpre_r, degp):
    bn = 1000
    grid = N // bn
    return pl.pallas_call(
        _tca_body,
        grid=(grid,),
        in_specs=[
            pl.BlockSpec((bn, D), lambda i: (i, 0)),
            pl.BlockSpec((D, D), lambda i: (0, 0)),
            pl.BlockSpec((1, D), lambda i: (0, 0)),
            pl.BlockSpec((NC, bn, 16), lambda i: (0, i, 0)),
        ],
        out_specs=[
            pl.BlockSpec((bn, D), lambda i: (i, 0)),
            pl.BlockSpec((bn, D), lambda i: (i, 0)),
            pl.BlockSpec((bn, 1), lambda i: (i, 0)),
        ],
        out_shape=[
            jax.ShapeDtypeStruct((N, D), jnp.float32),
            jax.ShapeDtypeStruct((N, D), jnp.float32),
            jax.ShapeDtypeStruct((N, 1), jnp.float32),
        ],
    )(x, wpret, bpre_r, degp)


def _tcb_body(h_ref, tp_ref, dinv_ref, wz0_ref, wz1_ref, wh0_ref, wh1_ref,
              bz_ref, bh_ref, wsum_ref, hr_ref, a_ref):
    h = h_ref[...]
    u = dinv_ref[...] * (tp_ref[0] + tp_ref[1])
    z = jax.nn.sigmoid(
        jnp.dot(h, wz0_ref[...], preferred_element_type=jnp.float32)
        - jnp.dot(u, wz1_ref[...], preferred_element_type=jnp.float32)
        + bz_ref[...])
    ht = jnp.tanh(
        jnp.dot(h, wh0_ref[...], preferred_element_type=jnp.float32)
        - jnp.dot(u, wh1_ref[...], preferred_element_type=jnp.float32)
        + bh_ref[...])
    hr = jnp.maximum((1.0 - z) * ht, 0.0)
    hr_ref[...] = hr
    a_ref[...] = hr * wsum_ref[...]


def _tc_gates(h, tp, dinv, wz0, wz1, wh0, wh1, bz_r, bh_r, wsum_r):
    bn = 1000
    grid = N // bn
    wspec = pl.BlockSpec((D, D), lambda i: (0, 0))
    bspec = pl.BlockSpec((1, D), lambda i: (0, 0))
    return pl.pallas_call(
        _tcb_body,
        grid=(grid,),
        in_specs=[
            pl.BlockSpec((bn, D), lambda i: (i, 0)),
            pl.BlockSpec((NC, bn, D), lambda i: (0, i, 0)),
            pl.BlockSpec((bn, 1), lambda i: (i, 0)),
            wspec, wspec, wspec, wspec, bspec, bspec, bspec,
        ],
        out_specs=[
            pl.BlockSpec((bn, D), lambda i: (i, 0)),
            pl.BlockSpec((bn, D), lambda i: (i, 0)),
        ],
        out_shape=[
            jax.ShapeDtypeStruct((N, D), jnp.float32),
            jax.ShapeDtypeStruct((N, D), jnp.float32),
        ],
    )(h, tp, dinv, wz0, wz1, wh0, wh1, bz_r, bh_r, wsum_r)


def kernel(x, edge_index, edge_label_index, Wpre, bpre,
           xz_W0, xz_W1, xz_b, hz_W0, hz_W1, hz_b,
           xr_W0, xr_W1, xr_b, hr_W0, hr_W1, hr_b,
           xh_W0, xh_W1, xh_b, hh_W0, hh_W1, hh_b,
           Wpost, bpost):
    src = edge_index[0]
    dst = edge_index[1]
    dscan = (dst // RNG) | (src << 4) | ((dst % RNG) << 18)
    sscan = (src // RNG) | ((src % RNG) << 4)

    degp, dlist, dcnt = _sc_route(dscan, sscan)
    h, hs, dinv = _tc_pre(x, Wpre.T, bpre[None, :], degp[:, :N, :])
    tp = _sc_accum(hs, dlist, dcnt).reshape(NC, NP, D)[:, :N, :]
    hrelu, a = _tc_gates(
        h, tp, dinv, xz_W0, xz_W1, xh_W0, xh_W1,
        (xz_b + hz_b)[None, :], (xh_b + hh_b)[None, :],
        (Wpost[0] + Wpost[1])[None, :])

    eli = jnp.concatenate(
        [edge_label_index,
         jnp.zeros((2, ELP - EL), dtype=edge_label_index.dtype)], axis=1)
    s_r = eli[0].reshape(NW, SCH, SEC)
    d_r = eli[1].reshape(NW, SCH, SEC)
    bsum_arr = jnp.full((16,), bpost[0] + bpost[1], dtype=jnp.float32)

    scores = _sc_score(a, hrelu, s_r, d_r, bsum_arr)
    return scores[:EL]


# speedup vs baseline: 1.3759x; 1.0900x over previous
import functools

import jax
import jax.numpy as jnp
from jax import lax
from jax.experimental import pallas as pl
from jax.experimental.pallas import tpu as pltpu
from jax.experimental.pallas import tpu_sc as plsc

N = 10000
D = 128
E = 320000
EL = 100000

NC = 2
NS = 16
NW = NC * NS

NP = 10240
RNG = NP // NS
DUMMYDL = RNG
ACC2R = RNG + 8

EH = E // NC
SCCH = 2000
NSC = EH // SCCH
ECAP = EH
LBUF = 4096
B2 = 128

SEC = 128
SCH = 26
SPT = SEC * SCH
ELP = NW * SPT

_MESH = plsc.VectorSubcoreMesh(core_axis_name="c", subcore_axis_name="s")
_SC_PARAMS = pltpu.CompilerParams(needs_layout_passes=False)


def _wid():
    return lax.axis_index("s") * NC + lax.axis_index("c")


@functools.partial(
    pl.kernel,
    out_type=[
        jax.ShapeDtypeStruct((NC, NP, 16), jnp.float32),
        jax.ShapeDtypeStruct((NW * ECAP,), jnp.int32),
        jax.ShapeDtypeStruct((NW * 16,), jnp.int32),
    ],
    mesh=_MESH,
    compiler_params=_SC_PARAMS,
    scratch_types=[
        pltpu.VMEM((SCCH,), jnp.int32),
        pltpu.VMEM((SCCH,), jnp.int32),
        pltpu.VMEM((SCCH,), jnp.int32),
        pltpu.VMEM((SCCH,), jnp.int32),
        pltpu.VMEM((LBUF,), jnp.int32),
        pltpu.VMEM((RNG, 16), jnp.float32),
        pltpu.VMEM((16,), jnp.int32),
        pltpu.VMEM((16,), jnp.int32),
        pltpu.SemaphoreType.DMA,
        pltpu.SemaphoreType.DMA,
    ],
)
def _sc_route(dscan_hbm, sscan_hbm, degp_hbm, dlist_hbm, dcnt_hbm,
              dbuf0, sbuf0, dbuf1, sbuf1, lbuf, acc1, cbuf, stg, sem0, sem1):
    c = lax.axis_index("c")
    s = lax.axis_index("s")
    wid = _wid()
    lane = lax.iota(jnp.int32, 16)
    ones = jnp.full((16,), 1.0, jnp.float32)

    def z1(i, carry):
        acc1[i, :] = jnp.zeros((16,), jnp.float32)
        return carry

    lax.fori_loop(0, RNG, z1, 0)

    base = c * EH
    lbase = wid * ECAP

    def start(k, dbuf, sbuf, sem):
        pltpu.async_copy(dscan_hbm.at[pl.ds(base + k * SCCH, SCCH)], dbuf, sem)
        pltpu.async_copy(sscan_hbm.at[pl.ds(base + k * SCCH, SCCH)], sbuf, sem)

    def drain(dbuf, sbuf, sem):
        pltpu.make_async_copy(dscan_hbm.at[pl.ds(0, SCCH)], dbuf, sem).wait()
        pltpu.make_async_copy(sscan_hbm.at[pl.ds(0, SCCH)], sbuf, sem).wait()

    def chunk(carry, dbuf, sbuf):
        ptr, fo, tc, tail = carry

        def vec(i, c2):
            p, tc, tail = c2
            vd = dbuf[pl.ds(i * 16, 16)]
            vs = sbuf[pl.ds(i * 16, 16)]
            m2 = (vs & 15) == s
            plsc.addupdate_scatter(acc1, [vs >> 4, lane], ones, mask=m2)
            m1 = (vd & 15) == s
            cnt = plsc.all_reduce_population_count(m1)[0]
            plsc.store_compressed(stg.at[pl.ds(0, 16)], vd >> 4, mask=m1)
            cv = stg[pl.ds(0, 16)]
            rot = cv.at[(lane - tc) & 15].get(mode="promise_in_bounds")
            merged = jnp.where(lane >= tc, rot, tail)
            newfill = tc + cnt

            @pl.when(newfill >= 16)
            def _():
                lbuf[pl.ds(pl.multiple_of(p, 16), 16)] = merged

            p = p + jnp.where(newfill >= 16, 16, 0)
            tail = jnp.where(newfill >= 16, rot, merged)
            return p, newfill & 15, tail

        ptr, tc, tail = lax.fori_loop(0, SCCH // 16, vec, (ptr, tc, tail))

        flushed = jnp.where(ptr >= 2048, 1, 0)

        @pl.when(flushed == 1)
        def _():
            pltpu.sync_copy(lbuf.at[pl.ds(0, 2048)],
                            dlist_hbm.at[pl.ds(pl.multiple_of(lbase + fo, 128),
                                               2048)])
            for i in range(125):
                lbuf[pl.ds(i * 16, 16)] = lbuf[pl.ds(2048 + i * 16, 16)]

        return ptr - flushed * 2048, fo + flushed * 2048, tc, tail

    zero16 = jnp.zeros((16,), jnp.int32)
    start(0, dbuf0, sbuf0, sem0)

    def pair(p, carry):
        ka = 2 * p
        start(ka + 1, dbuf1, sbuf1, sem1)
        drain(dbuf0, sbuf0, sem0)
        carry = chunk(carry, dbuf0, sbuf0)

        @pl.when(p < NSC // 2 - 1)
        def _():
            start(ka + 2, dbuf0, sbuf0, sem0)

        drain(dbuf1, sbuf1, sem1)
        carry = chunk(carry, dbuf1, sbuf1)
        return carry

    ptr, fo, tc, tail = lax.fori_loop(0, NSC // 2, pair, (0, 0, 0, zero16))

    dummy = jnp.full((16,), DUMMYDL << 14, jnp.int32)

    @pl.when(tc > 0)
    def _():
        lbuf[pl.ds(pl.multiple_of(ptr, 16), 16)] =             jnp.where(lane < tc, tail, dummy)

    ptr = ptr + jnp.where(tc > 0, 16, 0)

    def pad(i, p):
        rem = p & 127

        @pl.when(rem != 0)
        def _():
            lbuf[pl.ds(pl.multiple_of(p, 16), 16)] = dummy

        return p + jnp.where(rem != 0, 16, 0)

    ptr = lax.fori_loop(0, 7, pad, ptr)

    nb = ptr >> 7

    def fl(i, carry):
        @pl.when(i < nb)
        def _():
            pltpu.sync_copy(
                lbuf.at[pl.ds(i * 128, 128)],
                dlist_hbm.at[pl.ds(pl.multiple_of(lbase + fo + i * 128, 128),
                                   128)])
        return carry

    lax.fori_loop(0, 32, fl, 0)

    total = fo + ptr
    cbuf[pl.ds(0, 16)] = jnp.zeros((16,), jnp.int32) + total
    pltpu.sync_copy(cbuf, dcnt_hbm.at[pl.ds(wid * 16, 16)])
    pltpu.sync_copy(acc1, degp_hbm.at[c].at[pl.ds(s * RNG, RNG)])


@functools.partial(
    pl.kernel,
    out_type=jax.ShapeDtypeStruct((NC * NP * D,), jnp.float32),
    mesh=_MESH,
    compiler_params=_SC_PARAMS,
    scratch_types=[
        pltpu.VMEM((B2, D), jnp.float32),
        pltpu.VMEM((B2, D), jnp.float32),
        pltpu.VMEM((B2,), jnp.int32),
        pltpu.VMEM((B2,), jnp.int32),
        pltpu.VMEM((B2,), jnp.int32),
        pltpu.VMEM((B2,), jnp.int32),
        pltpu.VMEM((ACC2R * D,), jnp.float32),
        pltpu.VMEM((16,), jnp.int32),
        pltpu.SemaphoreType.DMA,
        pltpu.SemaphoreType.DMA,
    ],
)
def _sc_accum(hs_hbm, dlist_hbm, dcnt_hbm, out_hbm,
              rows0, rows1, sidx0, sidx1, pbuf0, pbuf1, acc2, cbuf,
              sem0, sem1):
    c = lax.axis_index("c")
    s = lax.axis_index("s")
    wid = _wid()

    def z2(i, carry):
        for k in range(D // 16):
            acc2[pl.ds(i * D + k * 16, 16)] = jnp.zeros((16,), jnp.float32)
        return carry

    lax.fori_loop(0, ACC2R, z2, 0)

    pltpu.sync_copy(dcnt_hbm.at[pl.ds(wid * 16, 16)], cbuf)
    total = cbuf[pl.ds(0, 16)][0]
    nb = total >> 7
    lbase = wid * ECAP

    lane = lax.iota(jnp.int32, 16)
    cols = [k * 16 + lane for k in range(D // 16)]

    def fetch(b, pbuf, sidx, rows, sem):
        off = pl.multiple_of(lbase + b * B2, 128)
        pltpu.sync_copy(dlist_hbm.at[pl.ds(off, B2)], pbuf)
        for i in range(B2 // 16):
            sidx[pl.ds(i * 16, 16)] = pbuf[pl.ds(i * 16, 16)] & 0x3FFF
        pltpu.async_copy(hs_hbm.at[sidx], rows, sem)

    def drain(sidx, rows, sem):
        pltpu.make_async_copy(hs_hbm.at[sidx], rows, sem).wait()

    def process(pbuf, rows):
        def group(g, c2):
            wbv = (pbuf[pl.ds(g * 16, 16)] >> 14) * D

            def edge(ee, c3):
                wb = wbv.at[jnp.zeros((16,), jnp.int32) + ee].get(
                    mode="promise_in_bounds")
                e = g * 16 + ee
                for k in range(D // 16):
                    plsc.addupdate_scatter(
                        acc2, [wb + cols[k]], rows[e, pl.ds(k * 16, 16)])
                return c3

            lax.fori_loop(0, 16, edge, 0)
            return c2

        lax.fori_loop(0, B2 // 16, group, 0)

    @pl.when(nb > 0)
    def _():
        fetch(0, pbuf0, sidx0, rows0, sem0)

    def pair(i, carry):
        ja = 2 * i

        @pl.when(ja + 1 < nb)
        def _():
            fetch(ja + 1, pbuf1, sidx1, rows1, sem1)

        drain(sidx0, rows0, sem0)
        process(pbuf0, rows0)

        @pl.when(ja + 2 < nb)
        def _():
            fetch(ja + 2, pbuf0, sidx0, rows0, sem0)

        @pl.when(ja + 1 < nb)
        def _():
            drain(sidx1, rows1, sem1)
            process(pbuf1, rows1)

        return carry

    lax.fori_loop(0, (nb + 1) >> 1, pair, 0)
    obase = (c * NP + s * RNG) * D
    pltpu.sync_copy(acc2.at[pl.ds(0, RNG * D)],
                    out_hbm.at[pl.ds(pl.multiple_of(obase, 128), RNG * D)])


@functools.partial(
    pl.kernel,
    out_type=jax.ShapeDtypeStruct((ELP,), jnp.float32),
    mesh=_MESH,
    compiler_params=_SC_PARAMS,
    scratch_types=[
        pltpu.VMEM((SCH, SEC), jnp.int32),
        pltpu.VMEM((SCH, SEC), jnp.int32),
        pltpu.VMEM((SEC, D), jnp.float32),
        pltpu.VMEM((SEC, D), jnp.float32),
        pltpu.VMEM((SEC, D), jnp.float32),
        pltpu.VMEM((SEC, D), jnp.float32),
        pltpu.VMEM((SPT,), jnp.float32),
        pltpu.VMEM((16,), jnp.float32),
        pltpu.SemaphoreType.DMA,
        pltpu.SemaphoreType.DMA,
    ],
)
def _sc_score(a_hbm, h_hbm, s_hbm, d_hbm, bsum_hbm, out_hbm,
              idx_s, idx_d, ra0, rb0, ra1, rb1, out_v, bsum_v, sem0, sem1):
    wid = _wid()
    pltpu.sync_copy(s_hbm.at[wid], idx_s)
    pltpu.sync_copy(d_hbm.at[wid], idx_d)
    pltpu.sync_copy(bsum_hbm, bsum_v)
    bsum = bsum_v[pl.ds(0, 16)]
    lane = lax.iota(jnp.int32, 16)

    def start(j, ra, rb, sem):
        pltpu.async_copy(a_hbm.at[idx_s.at[j]], ra, sem)
        pltpu.async_copy(h_hbm.at[idx_d.at[j]], rb, sem)

    def drain(ra, rb, sem):
        pltpu.make_async_copy(a_hbm.at[idx_s.at[0]], ra, sem).wait()
        pltpu.make_async_copy(h_hbm.at[idx_d.at[0]], rb, sem).wait()

    def process(j, ra, rb):
        def group(g, carry2):
            erow = g * 16 + lane
            acc = bsum
            for k in range(D):
                col = jnp.full((16,), k, jnp.int32)
                acc = acc + plsc.load_gather(ra, [erow, col]) * \
                    plsc.load_gather(rb, [erow, col])
            out_v[pl.ds(j * SEC + g * 16, 16)] = acc
            return carry2

        lax.fori_loop(0, SEC // 16, group, 0)

    start(0, ra0, rb0, sem0)

    def pair(i, carry):
        ja = 2 * i
        start(ja + 1, ra1, rb1, sem1)
        drain(ra0, rb0, sem0)
        process(ja, ra0, rb0)

        @pl.when(i < SCH // 2 - 1)
        def _():
            start(ja + 2, ra0, rb0, sem0)

        drain(ra1, rb1, sem1)
        process(ja + 1, ra1, rb1)
        return carry

    lax.fori_loop(0, SCH // 2, pair, 0)
    pltpu.sync_copy(out_v, out_hbm.at[pl.ds(wid * SPT, SPT)])


def _tca_body(x_ref, wpret_ref, bpre_ref, degp_ref, h_ref, hs_ref, dinv_ref):
    h = jnp.dot(x_ref[...], wpret_ref[...],
                preferred_element_type=jnp.float32) + bpre_ref[...]
    deg = jnp.sum(degp_ref[0] + degp_ref[1], axis=-1, keepdims=True)
    dinv = jnp.where(deg > 0, lax.rsqrt(deg), 0.0)
    h_ref[...] = h
    hs_ref[...] = h * dinv
    dinv_ref[...] = dinv


def _tc_pre(x, wpret, bpre_r, degp):
    bn = 1000
    grid = N // bn
    return pl.pallas_call(
        _tca_body,
        grid=(grid,),
        in_specs=[
            pl.BlockSpec((bn, D), lambda i: (i, 0)),
            pl.BlockSpec((D, D), lambda i: (0, 0)),
            pl.BlockSpec((1, D), lambda i: (0, 0)),
            pl.BlockSpec((NC, bn, 16), lambda i: (0, i, 0)),
        ],
        out_specs=[
            pl.BlockSpec((bn, D), lambda i: (i, 0)),
            pl.BlockSpec((bn, D), lambda i: (i, 0)),
            pl.BlockSpec((bn, 1), lambda i: (i, 0)),
        ],
        out_shape=[
            jax.ShapeDtypeStruct((N, D), jnp.float32),
            jax.ShapeDtypeStruct((N, D), jnp.float32),
            jax.ShapeDtypeStruct((N, 1), jnp.float32),
        ],
    )(x, wpret, bpre_r, degp)


def _tcb_body(h_ref, tp_ref, dinv_ref, wz0_ref, wz1_ref, wh0_ref, wh1_ref,
              bz_ref, bh_ref, wsum_ref, hr_ref, a_ref):
    h = h_ref[...]
    u = dinv_ref[...] * (tp_ref[0] + tp_ref[1])
    z = jax.nn.sigmoid(
        jnp.dot(h, wz0_ref[...], preferred_element_type=jnp.float32)
        - jnp.dot(u, wz1_ref[...], preferred_element_type=jnp.float32)
        + bz_ref[...])
    ht = jnp.tanh(
        jnp.dot(h, wh0_ref[...], preferred_element_type=jnp.float32)
        - jnp.dot(u, wh1_ref[...], preferred_element_type=jnp.float32)
        + bh_ref[...])
    hr = jnp.maximum((1.0 - z) * ht, 0.0)
    hr_ref[...] = hr
    a_ref[...] = hr * wsum_ref[...]


def _tc_gates(h, tp, dinv, wz0, wz1, wh0, wh1, bz_r, bh_r, wsum_r):
    bn = 1000
    grid = N // bn
    wspec = pl.BlockSpec((D, D), lambda i: (0, 0))
    bspec = pl.BlockSpec((1, D), lambda i: (0, 0))
    return pl.pallas_call(
        _tcb_body,
        grid=(grid,),
        in_specs=[
            pl.BlockSpec((bn, D), lambda i: (i, 0)),
            pl.BlockSpec((NC, bn, D), lambda i: (0, i, 0)),
            pl.BlockSpec((bn, 1), lambda i: (i, 0)),
            wspec, wspec, wspec, wspec, bspec, bspec, bspec,
        ],
        out_specs=[
            pl.BlockSpec((bn, D), lambda i: (i, 0)),
            pl.BlockSpec((bn, D), lambda i: (i, 0)),
        ],
        out_shape=[
            jax.ShapeDtypeStruct((N, D), jnp.float32),
            jax.ShapeDtypeStruct((N, D), jnp.float32),
        ],
    )(h, tp, dinv, wz0, wz1, wh0, wh1, bz_r, bh_r, wsum_r)


def kernel(x, edge_index, edge_label_index, Wpre, bpre,
           xz_W0, xz_W1, xz_b, hz_W0, hz_W1, hz_b,
           xr_W0, xr_W1, xr_b, hr_W0, hr_W1, hr_b,
           xh_W0, xh_W1, xh_b, hh_W0, hh_W1, hh_b,
           Wpost, bpost):
    src = edge_index[0]
    dst = edge_index[1]
    dscan = (dst // RNG) | (src << 4) | ((dst % RNG) << 18)
    sscan = (src // RNG) | ((src % RNG) << 4)

    degp, dlist, dcnt = _sc_route(dscan, sscan)
    h, hs, dinv = _tc_pre(x, Wpre.T, bpre[None, :], degp[:, :N, :])
    tp = _sc_accum(hs, dlist, dcnt).reshape(NC, NP, D)[:, :N, :]
    hrelu, a = _tc_gates(
        h, tp, dinv, xz_W0, xz_W1, xh_W0, xh_W1,
        (xz_b + hz_b)[None, :], (xh_b + hh_b)[None, :],
        (Wpost[0] + Wpost[1])[None, :])

    eli = jnp.concatenate(
        [edge_label_index,
         jnp.zeros((2, ELP - EL), dtype=edge_label_index.dtype)], axis=1)
    s_r = eli[0].reshape(NW, SCH, SEC)
    d_r = eli[1].reshape(NW, SCH, SEC)
    bsum_arr = jnp.full((16,), bpost[0] + bpost[1], dtype=jnp.float32)

    scores = _sc_score(a, hrelu, s_r, d_r, bsum_arr)
    return scores[:EL]


# speedup vs baseline: 1.5652x; 1.1376x over previous
import functools

import jax
import jax.numpy as jnp
from jax import lax
from jax.experimental import pallas as pl
from jax.experimental.pallas import tpu as pltpu
from jax.experimental.pallas import tpu_sc as plsc

N = 10000
D = 128
E = 320000
EL = 100000

NC = 2
NS = 16
NW = NC * NS

NP = 10240
RNG = NP // NS
DUMMYDL = RNG
ACC2R = RNG + 8

EH = E // NC
SCCH = 2000
NSC = EH // SCCH
ECAP = EH
LBUF = 4096
B2 = 128

SEC = 128
SCH = 26
SPT = SEC * SCH
ELP = NW * SPT

_MESH = plsc.VectorSubcoreMesh(core_axis_name="c", subcore_axis_name="s")
_SC_PARAMS = pltpu.CompilerParams(needs_layout_passes=False)


def _wid():
    return lax.axis_index("s") * NC + lax.axis_index("c")


@functools.partial(
    pl.kernel,
    out_type=[
        jax.ShapeDtypeStruct((NC, NP, 16), jnp.float32),
        jax.ShapeDtypeStruct((NW * ECAP,), jnp.int32),
        jax.ShapeDtypeStruct((NW * 16,), jnp.int32),
    ],
    mesh=_MESH,
    compiler_params=_SC_PARAMS,
    scratch_types=[
        pltpu.VMEM((SCCH,), jnp.int32),
        pltpu.VMEM((SCCH,), jnp.int32),
        pltpu.VMEM((SCCH,), jnp.int32),
        pltpu.VMEM((SCCH,), jnp.int32),
        pltpu.VMEM((LBUF,), jnp.int32),
        pltpu.VMEM((RNG, 16), jnp.float32),
        pltpu.VMEM((16,), jnp.int32),
        pltpu.VMEM((16,), jnp.int32),
        pltpu.SemaphoreType.DMA,
        pltpu.SemaphoreType.DMA,
    ],
)
def _sc_route(dscan_hbm, sscan_hbm, degp_hbm, dlist_hbm, dcnt_hbm,
              dbuf0, sbuf0, dbuf1, sbuf1, lbuf, acc1, cbuf, stg, sem0, sem1):
    c = lax.axis_index("c")
    s = lax.axis_index("s")
    wid = _wid()
    lane = lax.iota(jnp.int32, 16)
    ones = jnp.full((16,), 1.0, jnp.float32)

    def z1(i, carry):
        acc1[i, :] = jnp.zeros((16,), jnp.float32)
        return carry

    lax.fori_loop(0, RNG, z1, 0)

    base = c * EH
    lbase = wid * ECAP

    def start(k, dbuf, sbuf, sem):
        pltpu.async_copy(dscan_hbm.at[pl.ds(base + k * SCCH, SCCH)], dbuf, sem)
        pltpu.async_copy(sscan_hbm.at[pl.ds(base + k * SCCH, SCCH)], sbuf, sem)

    def drain(dbuf, sbuf, sem):
        pltpu.make_async_copy(dscan_hbm.at[pl.ds(0, SCCH)], dbuf, sem).wait()
        pltpu.make_async_copy(sscan_hbm.at[pl.ds(0, SCCH)], sbuf, sem).wait()

    def chunk(carry, dbuf, sbuf):
        ptr, fo, tc, tail = carry

        def vec(i, c2):
            p, tc, tail = c2
            vd = dbuf[pl.ds(i * 16, 16)]
            vs = sbuf[pl.ds(i * 16, 16)]
            m2 = (vs & 15) == s
            plsc.addupdate_scatter(acc1, [vs >> 4, lane], ones, mask=m2)
            m1 = (vd & 15) == s
            cnt = plsc.all_reduce_population_count(m1)[0]
            plsc.store_compressed(stg.at[pl.ds(0, 16)], vd >> 4, mask=m1)
            cv = stg[pl.ds(0, 16)]
            rot = cv.at[(lane - tc) & 15].get(mode="promise_in_bounds")
            merged = jnp.where(lane >= tc, rot, tail)
            newfill = tc + cnt

            @pl.when(newfill >= 16)
            def _():
                lbuf[pl.ds(pl.multiple_of(p, 16), 16)] = merged

            p = p + jnp.where(newfill >= 16, 16, 0)
            tail = jnp.where(newfill >= 16, rot, merged)
            return p, newfill & 15, tail

        ptr, tc, tail = lax.fori_loop(0, SCCH // 16, vec, (ptr, tc, tail))

        flushed = jnp.where(ptr >= 2048, 1, 0)

        @pl.when(flushed == 1)
        def _():
            pltpu.sync_copy(lbuf.at[pl.ds(0, 2048)],
                            dlist_hbm.at[pl.ds(pl.multiple_of(lbase + fo, 128),
                                               2048)])
            for i in range(125):
                lbuf[pl.ds(i * 16, 16)] = lbuf[pl.ds(2048 + i * 16, 16)]

        return ptr - flushed * 2048, fo + flushed * 2048, tc, tail

    zero16 = jnp.zeros((16,), jnp.int32)
    start(0, dbuf0, sbuf0, sem0)

    def pair(p, carry):
        ka = 2 * p
        start(ka + 1, dbuf1, sbuf1, sem1)
        drain(dbuf0, sbuf0, sem0)
        carry = chunk(carry, dbuf0, sbuf0)

        @pl.when(p < NSC // 2 - 1)
        def _():
            start(ka + 2, dbuf0, sbuf0, sem0)

        drain(dbuf1, sbuf1, sem1)
        carry = chunk(carry, dbuf1, sbuf1)
        return carry

    ptr, fo, tc, tail = lax.fori_loop(0, NSC // 2, pair, (0, 0, 0, zero16))

    dummy = jnp.full((16,), DUMMYDL << 14, jnp.int32)

    @pl.when(tc > 0)
    def _():
        lbuf[pl.ds(pl.multiple_of(ptr, 16), 16)] =             jnp.where(lane < tc, tail, dummy)

    ptr = ptr + jnp.where(tc > 0, 16, 0)

    def pad(i, p):
        rem = p & 127

        @pl.when(rem != 0)
        def _():
            lbuf[pl.ds(pl.multiple_of(p, 16), 16)] = dummy

        return p + jnp.where(rem != 0, 16, 0)

    ptr = lax.fori_loop(0, 7, pad, ptr)

    nb = ptr >> 7

    def fl(i, carry):
        @pl.when(i < nb)
        def _():
            pltpu.sync_copy(
                lbuf.at[pl.ds(i * 128, 128)],
                dlist_hbm.at[pl.ds(pl.multiple_of(lbase + fo + i * 128, 128),
                                   128)])
        return carry

    lax.fori_loop(0, 32, fl, 0)

    total = fo + ptr
    cbuf[pl.ds(0, 16)] = jnp.zeros((16,), jnp.int32) + total
    pltpu.sync_copy(cbuf, dcnt_hbm.at[pl.ds(wid * 16, 16)])
    pltpu.sync_copy(acc1, degp_hbm.at[c].at[pl.ds(s * RNG, RNG)])


@functools.partial(
    pl.kernel,
    out_type=jax.ShapeDtypeStruct((NC * NP * D,), jnp.float32),
    mesh=_MESH,
    compiler_params=_SC_PARAMS,
    scratch_types=[
        pltpu.VMEM((B2, D), jnp.float32),
        pltpu.VMEM((B2, D), jnp.float32),
        pltpu.VMEM((B2,), jnp.int32),
        pltpu.VMEM((B2,), jnp.int32),
        pltpu.VMEM((B2,), jnp.int32),
        pltpu.VMEM((B2,), jnp.int32),
        pltpu.VMEM((ACC2R * D,), jnp.float32),
        pltpu.VMEM((16,), jnp.int32),
        pltpu.SemaphoreType.DMA,
        pltpu.SemaphoreType.DMA,
    ],
)
def _sc_accum(hs_hbm, dlist_hbm, dcnt_hbm, out_hbm,
              rows0, rows1, sidx0, sidx1, pbuf0, pbuf1, acc2, cbuf,
              sem0, sem1):
    c = lax.axis_index("c")
    s = lax.axis_index("s")
    wid = _wid()

    def z2(i, carry):
        for k in range(D // 16):
            acc2[pl.ds(i * D + k * 16, 16)] = jnp.zeros((16,), jnp.float32)
        return carry

    lax.fori_loop(0, ACC2R, z2, 0)

    pltpu.sync_copy(dcnt_hbm.at[pl.ds(wid * 16, 16)], cbuf)
    total = cbuf[pl.ds(0, 16)][0]
    nb = total >> 7
    lbase = wid * ECAP

    lane = lax.iota(jnp.int32, 16)
    cols = [k * 16 + lane for k in range(D // 16)]

    def fetch(b, pbuf, sidx, rows, sem):
        off = pl.multiple_of(lbase + b * B2, 128)
        pltpu.sync_copy(dlist_hbm.at[pl.ds(off, B2)], pbuf)
        for i in range(B2 // 16):
            sidx[pl.ds(i * 16, 16)] = pbuf[pl.ds(i * 16, 16)] & 0x3FFF
        pltpu.async_copy(hs_hbm.at[sidx], rows, sem)

    def drain(sidx, rows, sem):
        pltpu.make_async_copy(hs_hbm.at[sidx], rows, sem).wait()

    def process(pbuf, rows):
        def group(g, c2):
            wbv = (pbuf[pl.ds(g * 16, 16)] >> 14) * D

            def edge(ee, c3):
                wb = wbv.at[jnp.zeros((16,), jnp.int32) + ee].get(
                    mode="promise_in_bounds")
                e = g * 16 + ee
                for k in range(D // 16):
                    plsc.addupdate_scatter(
                        acc2, [wb + cols[k]], rows[e, pl.ds(k * 16, 16)])
                return c3

            lax.fori_loop(0, 16, edge, 0)
            return c2

        lax.fori_loop(0, B2 // 16, group, 0)

    @pl.when(nb > 0)
    def _():
        fetch(0, pbuf0, sidx0, rows0, sem0)

    def pair(i, carry):
        ja = 2 * i

        @pl.when(ja + 1 < nb)
        def _():
            fetch(ja + 1, pbuf1, sidx1, rows1, sem1)

        drain(sidx0, rows0, sem0)
        process(pbuf0, rows0)

        @pl.when(ja + 2 < nb)
        def _():
            fetch(ja + 2, pbuf0, sidx0, rows0, sem0)

        @pl.when(ja + 1 < nb)
        def _():
            drain(sidx1, rows1, sem1)
            process(pbuf1, rows1)

        return carry

    lax.fori_loop(0, (nb + 1) >> 1, pair, 0)
    obase = (c * NP + s * RNG) * D
    pltpu.sync_copy(acc2.at[pl.ds(0, RNG * D)],
                    out_hbm.at[pl.ds(pl.multiple_of(obase, 128), RNG * D)])


@functools.partial(
    pl.kernel,
    out_type=jax.ShapeDtypeStruct((ELP,), jnp.float32),
    mesh=_MESH,
    compiler_params=_SC_PARAMS,
    scratch_types=[
        pltpu.VMEM((SCH, SEC), jnp.int32),
        pltpu.VMEM((SCH, SEC), jnp.int32),
        pltpu.VMEM((SEC, D), jnp.float32),
        pltpu.VMEM((SEC, D), jnp.float32),
        pltpu.VMEM((SEC, D), jnp.float32),
        pltpu.VMEM((SEC, D), jnp.float32),
        pltpu.VMEM((SPT,), jnp.float32),
        pltpu.VMEM((16,), jnp.float32),
        pltpu.SemaphoreType.DMA,
        pltpu.SemaphoreType.DMA,
    ],
)
def _sc_score(a_hbm, h_hbm, s_hbm, d_hbm, bsum_hbm, out_hbm,
              idx_s, idx_d, ra0, rb0, ra1, rb1, out_v, bsum_v, sem0, sem1):
    wid = _wid()
    pltpu.sync_copy(s_hbm.at[wid], idx_s)
    pltpu.sync_copy(d_hbm.at[wid], idx_d)
    pltpu.sync_copy(bsum_hbm, bsum_v)
    bsum = bsum_v[pl.ds(0, 16)]
    lane = lax.iota(jnp.int32, 16)

    def start(j, ra, rb, sem):
        pltpu.async_copy(a_hbm.at[idx_s.at[j]], ra, sem)
        pltpu.async_copy(h_hbm.at[idx_d.at[j]], rb, sem)

    def drain(ra, rb, sem):
        pltpu.make_async_copy(a_hbm.at[idx_s.at[0]], ra, sem).wait()
        pltpu.make_async_copy(h_hbm.at[idx_d.at[0]], rb, sem).wait()

    def process(j, ra, rb):
        def group(g, carry2):
            res = bsum

            for ee in range(16):
                e = g * 16 + ee
                m = [ra[e, pl.ds(k * 16, 16)] * rb[e, pl.ds(k * 16, 16)]
                     for k in range(D // 16)]
                t = ((m[0] + m[1]) + (m[2] + m[3])) + \
                    ((m[4] + m[5]) + (m[6] + m[7]))
                res = jnp.where(lane == ee, jnp.sum(t) + bsum[0], res)

            out_v[pl.ds(j * SEC + g * 16, 16)] = res
            return carry2

        lax.fori_loop(0, SEC // 16, group, 0)

    start(0, ra0, rb0, sem0)

    def pair(i, carry):
        ja = 2 * i
        start(ja + 1, ra1, rb1, sem1)
        drain(ra0, rb0, sem0)
        process(ja, ra0, rb0)

        @pl.when(i < SCH // 2 - 1)
        def _():
            start(ja + 2, ra0, rb0, sem0)

        drain(ra1, rb1, sem1)
        process(ja + 1, ra1, rb1)
        return carry

    lax.fori_loop(0, SCH // 2, pair, 0)
    pltpu.sync_copy(out_v, out_hbm.at[pl.ds(wid * SPT, SPT)])


def _tca_body(x_ref, wpret_ref, bpre_ref, degp_ref, h_ref, hs_ref, dinv_ref):
    h = jnp.dot(x_ref[...], wpret_ref[...],
                preferred_element_type=jnp.float32) + bpre_ref[...]
    deg = jnp.sum(degp_ref[0] + degp_ref[1], axis=-1, keepdims=True)
    dinv = jnp.where(deg > 0, lax.rsqrt(deg), 0.0)
    h_ref[...] = h
    hs_ref[...] = h * dinv
    dinv_ref[...] = dinv


def _tc_pre(x, wpret, bpre_r, degp):
    bn = 1000
    grid = N // bn
    return pl.pallas_call(
        _tca_body,
        grid=(grid,),
        in_specs=[
            pl.BlockSpec((bn, D), lambda i: (i, 0)),
            pl.BlockSpec((D, D), lambda i: (0, 0)),
            pl.BlockSpec((1, D), lambda i: (0, 0)),
            pl.BlockSpec((NC, bn, 16), lambda i: (0, i, 0)),
        ],
        out_specs=[
            pl.BlockSpec((bn, D), lambda i: (i, 0)),
            pl.BlockSpec((bn, D), lambda i: (i, 0)),
            pl.BlockSpec((bn, 1), lambda i: (i, 0)),
        ],
        out_shape=[
            jax.ShapeDtypeStruct((N, D), jnp.float32),
            jax.ShapeDtypeStruct((N, D), jnp.float32),
            jax.ShapeDtypeStruct((N, 1), jnp.float32),
        ],
    )(x, wpret, bpre_r, degp)


def _tcb_body(h_ref, tp_ref, dinv_ref, wz0_ref, wz1_ref, wh0_ref, wh1_ref,
              bz_ref, bh_ref, wsum_ref, hr_ref, a_ref):
    h = h_ref[...]
    u = dinv_ref[...] * (tp_ref[0] + tp_ref[1])
    z = jax.nn.sigmoid(
        jnp.dot(h, wz0_ref[...], preferred_element_type=jnp.float32)
        - jnp.dot(u, wz1_ref[...], preferred_element_type=jnp.float32)
        + bz_ref[...])
    ht = jnp.tanh(
        jnp.dot(h, wh0_ref[...], preferred_element_type=jnp.float32)
        - jnp.dot(u, wh1_ref[...], preferred_element_type=jnp.float32)
        + bh_ref[...])
    hr = jnp.maximum((1.0 - z) * ht, 0.0)
    hr_ref[...] = hr
    a_ref[...] = hr * wsum_ref[...]


def _tc_gates(h, tp, dinv, wz0, wz1, wh0, wh1, bz_r, bh_r, wsum_r):
    bn = 1000
    grid = N // bn
    wspec = pl.BlockSpec((D, D), lambda i: (0, 0))
    bspec = pl.BlockSpec((1, D), lambda i: (0, 0))
    return pl.pallas_call(
        _tcb_body,
        grid=(grid,),
        in_specs=[
            pl.BlockSpec((bn, D), lambda i: (i, 0)),
            pl.BlockSpec((NC, bn, D), lambda i: (0, i, 0)),
            pl.BlockSpec((bn, 1), lambda i: (i, 0)),
            wspec, wspec, wspec, wspec, bspec, bspec, bspec,
        ],
        out_specs=[
            pl.BlockSpec((bn, D), lambda i: (i, 0)),
            pl.BlockSpec((bn, D), lambda i: (i, 0)),
        ],
        out_shape=[
            jax.ShapeDtypeStruct((N, D), jnp.float32),
            jax.ShapeDtypeStruct((N, D), jnp.float32),
        ],
    )(h, tp, dinv, wz0, wz1, wh0, wh1, bz_r, bh_r, wsum_r)


def kernel(x, edge_index, edge_label_index, Wpre, bpre,
           xz_W0, xz_W1, xz_b, hz_W0, hz_W1, hz_b,
           xr_W0, xr_W1, xr_b, hr_W0, hr_W1, hr_b,
           xh_W0, xh_W1, xh_b, hh_W0, hh_W1, hh_b,
           Wpost, bpost):
    src = edge_index[0]
    dst = edge_index[1]
    dscan = (dst // RNG) | (src << 4) | ((dst % RNG) << 18)
    sscan = (src // RNG) | ((src % RNG) << 4)

    degp, dlist, dcnt = _sc_route(dscan, sscan)
    h, hs, dinv = _tc_pre(x, Wpre.T, bpre[None, :], degp[:, :N, :])
    tp = _sc_accum(hs, dlist, dcnt).reshape(NC, NP, D)[:, :N, :]
    hrelu, a = _tc_gates(
        h, tp, dinv, xz_W0, xz_W1, xh_W0, xh_W1,
        (xz_b + hz_b)[None, :], (xh_b + hh_b)[None, :],
        (Wpost[0] + Wpost[1])[None, :])

    eli = jnp.concatenate(
        [edge_label_index,
         jnp.zeros((2, ELP - EL), dtype=edge_label_index.dtype)], axis=1)
    s_r = eli[0].reshape(NW, SCH, SEC)
    d_r = eli[1].reshape(NW, SCH, SEC)
    bsum_arr = jnp.full((16,), bpost[0] + bpost[1], dtype=jnp.float32)

    scores = _sc_score(a, hrelu, s_r, d_r, bsum_arr)
    return scores[:EL]
